# baseline (speedup 1.0000x reference)
import functools

import jax
import jax.numpy as jnp
from jax.experimental import pallas as pl
from jax.experimental.pallas import tpu as pltpu

B, C, L = 16, 8, 65536
ROWS = 16
PAD = 128
TREND_K = 11
LOCAL_K = 5
TREND_H = 5
LOCAL_H = 2
TREND_SCALING = 0.6
DETAIL = 0.85
SPIKE_T = 3.5
SPIKE_D = 0.35
EPS = 1e-6
ITERS = 2


def _edge_matrices(h, k):
    i = jax.lax.broadcasted_iota(jnp.int32, (256, 128), 0)
    p = jax.lax.broadcasted_iota(jnp.int32, (256, 128), 1)
    inv_k = 1.0 / float(k)
    left = ((jnp.abs(i - p) <= h).astype(jnp.float32)
            + ((i >= 1) & (i <= h - p)).astype(jnp.float32)) * inv_k
    q = 128 + p
    right = ((jnp.abs(i - q) <= h).astype(jnp.float32)
             + ((i >= 510 - q - h) & (i <= 254)).astype(jnp.float32)) * inv_k
    return left, right


_dot = functools.partial(
    jax.lax.dot_general,
    dimension_numbers=(((1,), (0,)), ((), ())),
    preferred_element_type=jnp.float32,
    precision=jax.lax.Precision.HIGHEST)


def _denoise_body(x_ref, o_ref, ps_ref, w2_ref, s2_ref, rs_ref, bs_ref):
    l5l, l5r = _edge_matrices(LOCAL_H, LOCAL_K)
    t11l, t11r = _edge_matrices(TREND_H, TREND_K)
    bl_l = (1.0 - TREND_SCALING) * l5l + TREND_SCALING * t11l
    bl_r = (1.0 - TREND_SCALING) * l5r + TREND_SCALING * t11r

    zpad = jnp.zeros((ROWS, PAD), dtype=jnp.float32)
    for ref in (ps_ref, w2_ref, s2_ref):
        ref[:, 0:PAD] = zpad
        ref[:, PAD + L:PAD + L + PAD] = zpad

    def one_iter(dst_ref, dst_base):
        w2_ref[:, PAD:PAD + L] = (ps_ref[:, PAD:PAD + L]
                                  + ps_ref[:, PAD + 1:PAD + 1 + L])
        s2 = ((w2_ref[:, PAD - 2:PAD - 2 + L] + w2_ref[:, PAD:PAD + L])
              + ps_ref[:, PAD + 2:PAD + 2 + L])
        s2_ref[:, PAD:PAD + L] = s2
        s5 = (s2_ref[:, PAD - 3:PAD - 3 + L]
              + s2_ref[:, PAD + 3:PAD + 3 + L]) + ps_ref[:, PAD:PAD + L]
        rs_ref[:, 0:L] = ps_ref[:, PAD:PAD + L] - s2 * (1.0 / LOCAL_K)
        bs_ref[:, 0:L] = (s2 * ((1.0 - TREND_SCALING) / LOCAL_K)
                          + s5 * (TREND_SCALING / TREND_K))

        xl = ps_ref[:, PAD:PAD + 256]
        xr = ps_ref[:, PAD + L - 256:PAD + L]
        rs_ref[:, 0:128] = xl[:, :128] - _dot(xl, l5l)
        bs_ref[:, 0:128] = _dot(xl, bl_l)
        rs_ref[:, L - 128:L] = xr[:, 128:] - _dot(xr, l5r)
        bs_ref[:, L - 128:L] = _dot(xr, bl_r)

        r = rs_ref[:, 0:L]
        rsq = r * r
        sum_r = jnp.sum(r, axis=1, keepdims=True)
        sum_q = jnp.sum(rsq, axis=1, keepdims=True)
        var = (sum_q - sum_r * sum_r * (1.0 / L)) * (1.0 / (L - 1))
        scale = jnp.maximum(jnp.sqrt(jnp.maximum(var, 0.0)), EPS)
        thr2 = (scale * scale) * (SPIKE_T * SPIKE_T)
        rc = jnp.where(rsq > thr2, r * (DETAIL * SPIKE_D), r * DETAIL)
        dst_ref[:, dst_base:dst_base + L] = bs_ref[:, 0:L] + rc

    ps_ref[:, PAD:PAD + L] = x_ref[...]
    for it in range(ITERS):
        if it < ITERS - 1:
            one_iter(ps_ref, PAD)
        else:
            one_iter(o_ref, 0)


@jax.jit
def kernel(x):
    xf = x.astype(jnp.float32).reshape(B * C, L)
    out = pl.pallas_call(
        _denoise_body,
        grid=(B * C // ROWS,),
        in_specs=[pl.BlockSpec((ROWS, L), lambda i: (i, 0))],
        out_specs=pl.BlockSpec((ROWS, L), lambda i: (i, 0)),
        out_shape=jax.ShapeDtypeStruct((B * C, L), jnp.float32),
        scratch_shapes=[pltpu.VMEM((ROWS, L + 2 * PAD), jnp.float32),
                        pltpu.VMEM((ROWS, L + 2 * PAD), jnp.float32),
                        pltpu.VMEM((ROWS, L + 2 * PAD), jnp.float32),
                        pltpu.VMEM((ROWS, L), jnp.float32),
                        pltpu.VMEM((ROWS, L), jnp.float32)],
        compiler_params=pltpu.CompilerParams(
            dimension_semantics=("parallel",)),
    )(xf)
    return out.reshape(B, C, L)


# speedup vs baseline: 1.0023x; 1.0023x over previous
import functools

import jax
import jax.numpy as jnp
from jax.experimental import pallas as pl
from jax.experimental.pallas import tpu as pltpu

B, C, L = 16, 8, 65536
ROWS = 8
PAD = 128
TREND_K = 11
LOCAL_K = 5
TREND_H = 5
LOCAL_H = 2
TREND_SCALING = 0.6
DETAIL = 0.85
SPIKE_T = 3.5
SPIKE_D = 0.35
EPS = 1e-6
ITERS = 2


def _edge_matrices(h, k):
    i = jax.lax.broadcasted_iota(jnp.int32, (256, 128), 0)
    p = jax.lax.broadcasted_iota(jnp.int32, (256, 128), 1)
    inv_k = 1.0 / float(k)
    left = ((jnp.abs(i - p) <= h).astype(jnp.float32)
            + ((i >= 1) & (i <= h - p)).astype(jnp.float32)) * inv_k
    q = 128 + p
    right = ((jnp.abs(i - q) <= h).astype(jnp.float32)
             + ((i >= 510 - q - h) & (i <= 254)).astype(jnp.float32)) * inv_k
    return left, right


_dot = functools.partial(
    jax.lax.dot_general,
    dimension_numbers=(((1,), (0,)), ((), ())),
    preferred_element_type=jnp.float32,
    precision=jax.lax.Precision.HIGHEST)


def _denoise_body(x_ref, o_ref, ps_ref, w2_ref, s2_ref, rs_ref, bs_ref):
    l5l, l5r = _edge_matrices(LOCAL_H, LOCAL_K)
    t11l, t11r = _edge_matrices(TREND_H, TREND_K)
    bl_l = (1.0 - TREND_SCALING) * l5l + TREND_SCALING * t11l
    bl_r = (1.0 - TREND_SCALING) * l5r + TREND_SCALING * t11r

    zpad = jnp.zeros((ROWS, PAD), dtype=jnp.float32)
    for ref in (ps_ref, w2_ref, s2_ref):
        ref[:, 0:PAD] = zpad
        ref[:, PAD + L:PAD + L + PAD] = zpad

    def one_iter(dst_ref, dst_base):
        w2_ref[:, PAD:PAD + L] = (ps_ref[:, PAD:PAD + L]
                                  + ps_ref[:, PAD + 1:PAD + 1 + L])
        s2 = ((w2_ref[:, PAD - 2:PAD - 2 + L] + w2_ref[:, PAD:PAD + L])
              + ps_ref[:, PAD + 2:PAD + 2 + L])
        s2_ref[:, PAD:PAD + L] = s2
        s5 = (s2_ref[:, PAD - 3:PAD - 3 + L]
              + s2_ref[:, PAD + 3:PAD + 3 + L]) + ps_ref[:, PAD:PAD + L]
        rs_ref[:, 0:L] = ps_ref[:, PAD:PAD + L] - s2 * (1.0 / LOCAL_K)
        bs_ref[:, 0:L] = (s2 * ((1.0 - TREND_SCALING) / LOCAL_K)
                          + s5 * (TREND_SCALING / TREND_K))

        xl = ps_ref[:, PAD:PAD + 256]
        xr = ps_ref[:, PAD + L - 256:PAD + L]
        rs_ref[:, 0:128] = xl[:, :128] - _dot(xl, l5l)
        bs_ref[:, 0:128] = _dot(xl, bl_l)
        rs_ref[:, L - 128:L] = xr[:, 128:] - _dot(xr, l5r)
        bs_ref[:, L - 128:L] = _dot(xr, bl_r)

        r = rs_ref[:, 0:L]
        rsq = r * r
        sum_r = jnp.sum(r, axis=1, keepdims=True)
        sum_q = jnp.sum(rsq, axis=1, keepdims=True)
        var = (sum_q - sum_r * sum_r * (1.0 / L)) * (1.0 / (L - 1))
        scale = jnp.maximum(jnp.sqrt(jnp.maximum(var, 0.0)), EPS)
        thr2 = (scale * scale) * (SPIKE_T * SPIKE_T)
        rc = jnp.where(rsq > thr2, r * (DETAIL * SPIKE_D), r * DETAIL)
        dst_ref[:, dst_base:dst_base + L] = bs_ref[:, 0:L] + rc

    ps_ref[:, PAD:PAD + L] = x_ref[...]
    for it in range(ITERS):
        if it < ITERS - 1:
            one_iter(ps_ref, PAD)
        else:
            one_iter(o_ref, 0)


@jax.jit
def kernel(x):
    xf = x.astype(jnp.float32).reshape(B * C, L)
    out = pl.pallas_call(
        _denoise_body,
        grid=(B * C // ROWS,),
        in_specs=[pl.BlockSpec((ROWS, L), lambda i: (i, 0))],
        out_specs=pl.BlockSpec((ROWS, L), lambda i: (i, 0)),
        out_shape=jax.ShapeDtypeStruct((B * C, L), jnp.float32),
        scratch_shapes=[pltpu.VMEM((ROWS, L + 2 * PAD), jnp.float32),
                        pltpu.VMEM((ROWS, L + 2 * PAD), jnp.float32),
                        pltpu.VMEM((ROWS, L + 2 * PAD), jnp.float32),
                        pltpu.VMEM((ROWS, L), jnp.float32),
                        pltpu.VMEM((ROWS, L), jnp.float32)],
        compiler_params=pltpu.CompilerParams(
            dimension_semantics=("parallel",)),
    )(xf)
    return out.reshape(B, C, L)
